# Initial kernel scaffold; baseline (speedup 1.0000x reference)
import jax
import jax.numpy as jnp
from jax.experimental import pallas as pl


def kernel(embeddings, sp_seg, edges):
    raise NotImplementedError("write your pallas kernel here")


# speedup vs baseline: 7.5892x; 7.5892x over previous
import jax
import jax.numpy as jnp
from jax import lax
from jax.experimental import pallas as pl
from jax.experimental.pallas import tpu as pltpu

_TAU = 0.1
_S = 32


def _nce_body(npix, nblk, emb_ref, seg_ref, e0_ref, e1_ref, t_ref, s_ref,
              sums_ref, counts_ref, means_ref, nam_ref, acc_ref):
    phase = pl.program_id(0)
    i = pl.program_id(1)

    @pl.when(jnp.logical_and(phase == 0, i == 0))
    def _init():
        sums_ref[...] = jnp.zeros_like(sums_ref)
        counts_ref[...] = jnp.zeros_like(counts_ref)
        acc_ref[0] = 0.0
        acc_ref[1] = 0.0

    eb = emb_ref[...]
    seg = seg_ref[0]
    iota_s = lax.broadcasted_iota(jnp.int32, (_S, seg.shape[-1]), 0)
    oh = (iota_s == seg).astype(jnp.float32)

    @pl.when(phase == 0)
    def _pass1():
        sums_ref[...] += lax.dot_general(
            oh, eb, (((1,), (1,)), ((), ())), preferred_element_type=jnp.float32)
        counts_ref[...] += jnp.sum(oh, axis=1, keepdims=True)

    @pl.when(jnp.logical_and(phase == 1, i == 0))
    def _means_and_edges():
        means = sums_ref[...] / counts_ref[...]
        means_ref[...] = means
        na = jnp.sqrt(jnp.sum(means * means, axis=1, keepdims=True))
        nam_ref[...] = na
        e0 = e0_ref[...]
        e1 = e1_ref[...]
        it = lax.broadcasted_iota(jnp.int32, (_S, e0.shape[-1]), 0)
        oh0 = (it == e0).astype(jnp.float32)
        oh1 = (it == e1).astype(jnp.float32)
        cnt_ij = lax.dot_general(
            oh0, oh1, (((1,), (1,)), ((), ())), preferred_element_type=jnp.float32)
        gram = lax.dot_general(
            means, means, (((1,), (1,)), ((), ())), preferred_element_type=jnp.float32)
        na_outer = lax.dot_general(
            na, na, (((1,), (1,)), ((), ())), preferred_element_type=jnp.float32)
        cos_ij = gram / jnp.maximum(na_outer, 1e-8) / _TAU
        acc_ref[1] += jnp.sum(cnt_ij * jnp.exp(cos_ij))

    @pl.when(phase == 1)
    def _pass2():
        means = means_ref[...]
        dots = lax.dot_general(
            means, eb, (((1,), (0,)), ((), ())), preferred_element_type=jnp.float32)
        dot_p = jnp.sum(dots * oh, axis=0)
        na_p = jnp.sum(nam_ref[...] * oh, axis=0)
        nb_p = jnp.sqrt(jnp.sum(eb * eb, axis=0))
        sim = dot_p / (jnp.maximum(na_p * nb_p, 1e-8) * _TAU)
        acc_ref[0] += jnp.sum(sim)
        acc_ref[1] += jnp.sum(jnp.exp(sim))

    @pl.when(jnp.logical_and(phase == 1, i == nblk - 1))
    def _fin():
        t_ref[0, 0] = acc_ref[0]
        s_ref[0, 0] = acc_ref[1]


def kernel(embeddings, sp_seg, edges):
    C = embeddings.shape[1]
    npix = embeddings.shape[2] * embeddings.shape[3]
    BK = 1024
    nblk = npix // BK
    emb = embeddings.reshape(C, npix)
    seg = sp_seg.reshape(nblk, 1, BK)
    e0 = edges[0:1, :]
    e1 = edges[1:2, :]

    import functools
    body = functools.partial(_nce_body, npix, nblk)
    t, s = pl.pallas_call(
        body,
        grid=(2, nblk),
        in_specs=[
            pl.BlockSpec((C, BK), lambda p, i: (0, i)),
            pl.BlockSpec((1, 1, BK), lambda p, i: (i, 0, 0)),
            pl.BlockSpec((1, edges.shape[1]), lambda p, i: (0, 0)),
            pl.BlockSpec((1, edges.shape[1]), lambda p, i: (0, 0)),
        ],
        out_specs=[
            pl.BlockSpec(memory_space=pltpu.SMEM),
            pl.BlockSpec(memory_space=pltpu.SMEM),
        ],
        out_shape=[
            jax.ShapeDtypeStruct((1, 1), jnp.float32),
            jax.ShapeDtypeStruct((1, 1), jnp.float32),
        ],
        scratch_shapes=[
            pltpu.VMEM((_S, C), jnp.float32),
            pltpu.VMEM((_S, 1), jnp.float32),
            pltpu.VMEM((_S, C), jnp.float32),
            pltpu.VMEM((_S, 1), jnp.float32),
            pltpu.SMEM((2,), jnp.float32),
        ],
        compiler_params=pltpu.CompilerParams(
            dimension_semantics=("arbitrary", "arbitrary"),
        ),
    )(emb, seg, e0, e1)
    return jnp.log(s[0, 0]) - t[0, 0] / jnp.float32(npix)


# speedup vs baseline: 16.0913x; 2.1203x over previous
import functools

import jax
import jax.numpy as jnp
from jax import lax
from jax.experimental import pallas as pl
from jax.experimental.pallas import tpu as pltpu

_TAU = 0.1
_S = 32


def _nce_body(npix, nblk, emb_ref, seg_ref, e0_ref, e1_ref, t_ref, s_ref,
              cache_ref, nbsq_ref, sums_ref, counts_ref, means_ref, nam_ref,
              acc_ref):
    phase = pl.program_id(0)
    i = pl.program_id(1)

    @pl.when(jnp.logical_and(phase == 0, i == 0))
    def _init():
        sums_ref[...] = jnp.zeros_like(sums_ref)
        counts_ref[...] = jnp.zeros_like(counts_ref)
        acc_ref[0] = 0.0
        acc_ref[1] = 0.0

    seg = seg_ref[0]
    iota_s = lax.broadcasted_iota(jnp.int32, (_S, seg.shape[-1]), 0)
    oh = (iota_s == seg).astype(jnp.float32)

    @pl.when(phase == 0)
    def _pass1():
        eb = emb_ref[...]
        cache_ref[i] = eb
        nbsq_ref[i] = jnp.sum(eb * eb, axis=0, keepdims=True)
        sums_ref[...] += lax.dot_general(
            oh, eb, (((1,), (1,)), ((), ())), preferred_element_type=jnp.float32)
        counts_ref[...] += jnp.sum(oh, axis=1, keepdims=True)

    @pl.when(jnp.logical_and(phase == 1, i == 0))
    def _means_and_edges():
        means = sums_ref[...] / counts_ref[...]
        means_ref[...] = means
        na = jnp.sqrt(jnp.sum(means * means, axis=1, keepdims=True))
        nam_ref[...] = na
        e0 = e0_ref[...]
        e1 = e1_ref[...]
        it = lax.broadcasted_iota(jnp.int32, (_S, e0.shape[-1]), 0)
        oh0 = (it == e0).astype(jnp.float32)
        oh1 = (it == e1).astype(jnp.float32)
        cnt_ij = lax.dot_general(
            oh0, oh1, (((1,), (1,)), ((), ())), preferred_element_type=jnp.float32)
        gram = lax.dot_general(
            means, means, (((1,), (1,)), ((), ())), preferred_element_type=jnp.float32)
        na_outer = lax.dot_general(
            na, na, (((1,), (1,)), ((), ())), preferred_element_type=jnp.float32)
        cos_ij = gram / jnp.maximum(na_outer, 1e-8) / _TAU
        acc_ref[1] += jnp.sum(cnt_ij * jnp.exp(cos_ij))

    @pl.when(phase == 1)
    def _pass2():
        eb = cache_ref[i]
        means = means_ref[...]
        dots = lax.dot_general(
            means, eb, (((1,), (0,)), ((), ())), preferred_element_type=jnp.float32)
        dot_p = jnp.sum(dots * oh, axis=0)
        na_p = jnp.sum(nam_ref[...] * oh, axis=0)
        nb_p = jnp.sqrt(nbsq_ref[i][0])
        sim = dot_p / (jnp.maximum(na_p * nb_p, 1e-8) * _TAU)
        acc_ref[0] += jnp.sum(sim)
        acc_ref[1] += jnp.sum(jnp.exp(sim))

    @pl.when(jnp.logical_and(phase == 1, i == nblk - 1))
    def _fin():
        t_ref[0, 0] = acc_ref[0]
        s_ref[0, 0] = acc_ref[1]


def kernel(embeddings, sp_seg, edges):
    C = embeddings.shape[1]
    npix = embeddings.shape[2] * embeddings.shape[3]
    BK = 7168
    nblk = npix // BK
    emb = embeddings.reshape(C, npix)
    seg = sp_seg.reshape(nblk, 1, BK)
    e0 = edges[0:1, :]
    e1 = edges[1:2, :]

    body = functools.partial(_nce_body, npix, nblk)
    t, s = pl.pallas_call(
        body,
        grid=(2, nblk),
        in_specs=[
            pl.BlockSpec((C, BK), lambda p, i: (0, i * (1 - p))),
            pl.BlockSpec((1, 1, BK), lambda p, i: (i, 0, 0)),
            pl.BlockSpec((1, edges.shape[1]), lambda p, i: (0, 0)),
            pl.BlockSpec((1, edges.shape[1]), lambda p, i: (0, 0)),
        ],
        out_specs=[
            pl.BlockSpec(memory_space=pltpu.SMEM),
            pl.BlockSpec(memory_space=pltpu.SMEM),
        ],
        out_shape=[
            jax.ShapeDtypeStruct((1, 1), jnp.float32),
            jax.ShapeDtypeStruct((1, 1), jnp.float32),
        ],
        scratch_shapes=[
            pltpu.VMEM((nblk, C, BK), jnp.float32),
            pltpu.VMEM((nblk, 1, BK), jnp.float32),
            pltpu.VMEM((_S, C), jnp.float32),
            pltpu.VMEM((_S, 1), jnp.float32),
            pltpu.VMEM((_S, C), jnp.float32),
            pltpu.VMEM((_S, 1), jnp.float32),
            pltpu.SMEM((2,), jnp.float32),
        ],
        compiler_params=pltpu.CompilerParams(
            dimension_semantics=("arbitrary", "arbitrary"),
        ),
    )(emb, seg, e0, e1)
    return jnp.log(s[0, 0]) - t[0, 0] / jnp.float32(npix)
